# baseline (speedup 1.0000x reference)
import functools

import jax
import jax.numpy as jnp
from jax import lax
from jax.experimental import pallas as pl
from jax.experimental.pallas import tpu as pltpu
from jax.experimental.pallas import tpu_sc as plsc

B = 16384
NUM = 13
NCAT = 26
V = 100000
D = 16
OUT = NUM + NCAT * D

NC = 2
NS = 16
L = 16
NW = NC * NS
RPW = B // NW
R = 64
NCHUNK = RPW // R
IDS = NCAT * R
GPG = 128
NGATH = IDS // GPG
PER = 16 * NUM


def _body(bn_hbm, ids_hbm, tbl_hbm, sc_hbm, bi_hbm, out_hbm,
          ids_v, idx_v, gath_v, rowbuf, nin_v, sc_v, bi_v, sem):
    wid = lax.axis_index("s") * NC + lax.axis_index("c")

    pltpu.sync_copy(sc_hbm, sc_v)
    pltpu.sync_copy(bi_hbm, bi_v)
    pltpu.sync_copy(ids_hbm.at[:, pl.ds(wid * RPW, RPW)], ids_v)

    iota = lax.iota(jnp.int32, L)

    def chunk(ci, _):
        base = wid * RPW + ci * R
        col = ci * R

        def build_f(f, _):
            fbase = f * V
            for g in range(R // L):
                idv = ids_v[f, pl.ds(col + g * L, L)]
                pos = (g * L + iota) * NCAT + f
                plsc.store_scatter(idx_v, [pos], idv + fbase)
            return 0
        lax.fori_loop(0, NCAT, build_f, 0)

        copies = []
        for g in range(NGATH):
            cp = pltpu.make_async_copy(
                tbl_hbm.at[idx_v.at[pl.ds(g * GPG, GPG)]],
                gath_v.at[pl.ds(g * GPG, GPG)],
                sem)
            cp.start()
            copies.append(cp)

        pltpu.sync_copy(bn_hbm.at[pl.ds(base * NUM, R * NUM)],
                        nin_v.at[pl.ds(0, R * NUM)])

        def nrow(r, _):
            x = nin_v[pl.ds(r * NUM, L)]
            off = (r % 16) * NUM
            rowbuf[pl.ds(r * OUT, L)] = x * sc_v[pl.ds(off, L)] + bi_v[pl.ds(off, L)]
            return 0
        lax.fori_loop(0, R, nrow, 0)

        for cp in copies:
            cp.wait()

        def fix_f(f, _):
            for g in range(R // L):
                idv = ids_v[f, pl.ds(col + g * L, L)]

                @pl.when(jnp.any(idv == 0))
                def _slow():
                    for j in range(L):
                        r = g * L + j

                        @pl.when(idv[j] == 0)
                        def _zero():
                            gath_v[r * NCAT + f, :] = jnp.zeros((L,), jnp.float32)
            return 0
        lax.fori_loop(0, NCAT, fix_f, 0)

        def arow(r, _):
            for k in range(NCAT):
                rowbuf[pl.ds(r * OUT + NUM + k * D, L)] = gath_v[r * NCAT + k, :]
            return 0
        lax.fori_loop(0, R, arow, 0)

        pltpu.sync_copy(rowbuf, out_hbm.at[pl.ds(base * OUT, R * OUT)])
        return 0

    lax.fori_loop(0, NCHUNK, chunk, 0)


@jax.jit
def kernel(batch_num, cat_ids, tables, num_mean, num_std):
    scale = (1.0 / num_std).reshape(NUM)
    bias = (-num_mean / num_std).reshape(NUM)
    sc_t = jnp.tile(scale, 16)
    bi_t = jnp.tile(bias, 16)

    mesh = plsc.VectorSubcoreMesh(core_axis_name="c", subcore_axis_name="s",
                                  num_cores=NC, num_subcores=NS)
    run = pl.kernel(
        _body,
        out_type=jax.ShapeDtypeStruct((B * OUT,), jnp.float32),
        mesh=mesh,
        scratch_types=[
            pltpu.VMEM((NCAT, RPW), jnp.int32),
            pltpu.VMEM((IDS,), jnp.int32),
            pltpu.VMEM((IDS, D), jnp.float32),
            pltpu.VMEM((R * OUT,), jnp.float32),
            pltpu.VMEM((R * NUM + L,), jnp.float32),
            pltpu.VMEM((PER,), jnp.float32),
            pltpu.VMEM((PER,), jnp.float32),
            pltpu.SemaphoreType.DMA,
        ],
        compiler_params=pltpu.CompilerParams(use_tc_tiling_on_sc=False,
                                             needs_layout_passes=False),
    )
    out = run(batch_num.reshape(B * NUM),
              cat_ids,
              tables.reshape(NCAT * V, D),
              sc_t, bi_t)
    return out.reshape(B, OUT)


# speedup vs baseline: 4.4158x; 4.4158x over previous
import jax
import jax.numpy as jnp
from jax import lax
from jax.experimental import pallas as pl
from jax.experimental.pallas import tpu as pltpu
from jax.experimental.pallas import tpu_sc as plsc

B = 16384
NUM = 13
NCAT = 26
V = 100000
D = 16
OUT = NUM + NCAT * D

NC = 2
NS = 16
L = 16
NW = NC * NS
RPW = (NCAT * D) // NW
H = B // 2
CH = H // L


def _body(bn_hbm, ids_hbm, t2_hbm, sc_hbm, bi_hbm, out_hbm,
          idx_v, row_v, gbuf, sc_v, bi_v, flag_v):
    wid = lax.axis_index("s") * NC + lax.axis_index("c")
    pltpu.sync_copy(sc_hbm, sc_v)
    pltpu.sync_copy(bi_hbm, bi_v)

    @pl.when(wid < NUM)
    def _num():
        widv = jnp.full((L,), 0, jnp.int32) + wid
        s16 = plsc.load_gather(sc_v, [widv])
        b16 = plsc.load_gather(bi_v, [widv])
        for h in range(2):
            pltpu.sync_copy(bn_hbm.at[wid, pl.ds(h * H, H)], gbuf)

            def nchunk(k, _):
                x = gbuf[pl.ds(k * L, L)]
                gbuf[pl.ds(k * L, L)] = x * s16 + b16
                return 0
            lax.fori_loop(0, CH, nchunk, 0)
            pltpu.sync_copy(gbuf, out_hbm.at[wid, pl.ds(h * H, H)])

    def stage_field(f):
        pltpu.sync_copy(ids_hbm.at[f], idx_v)

        def scan(k, acc):
            iv = idx_v[pl.ds(k * L, L)]
            return jnp.where(jnp.any(iv == 0), jnp.int32(1), acc)
        hz = lax.fori_loop(0, B // L, scan, jnp.int32(0))
        flag_v[pl.ds(0, L)] = jnp.zeros((L,), jnp.int32) + hz

    r0 = wid * RPW
    for j in range(RPW):
        r = r0 + j
        f = r // D
        d = r % D
        if j == 0:
            stage_field(f)
        else:
            fp = (r0 + j - 1) // D

            @pl.when(f != fp)
            def _restage():
                stage_field(f)

        pltpu.sync_copy(t2_hbm.at[f, d], row_v)
        fv = flag_v[pl.ds(0, L)]
        hz = fv[0] != 0
        c = r + NUM
        for h in range(2):
            def gchunk(k, _):
                iv = idx_v[pl.ds(h * H + k * L, L)]
                gbuf[pl.ds(k * L, L)] = plsc.load_gather(row_v, [iv])
                return 0
            lax.fori_loop(0, CH, gchunk, 0)

            @pl.when(hz)
            def _fix():
                def fchunk(k, _):
                    iv = idx_v[pl.ds(h * H + k * L, L)]
                    v = gbuf[pl.ds(k * L, L)]
                    gbuf[pl.ds(k * L, L)] = jnp.where(iv == 0, 0.0, v)
                    return 0
                lax.fori_loop(0, CH, fchunk, 0)
            pltpu.sync_copy(gbuf, out_hbm.at[c, pl.ds(h * H, H)])


@jax.jit
def kernel(batch_num, cat_ids, tables, num_mean, num_std):
    t2 = jnp.transpose(tables, (0, 2, 1))
    bn_t = batch_num.T
    scale = 1.0 / num_std.reshape(NUM)
    sc = jnp.pad(scale, (0, L - NUM))
    bi = jnp.pad(-num_mean.reshape(NUM) * scale, (0, L - NUM))

    mesh = plsc.VectorSubcoreMesh(core_axis_name="c", subcore_axis_name="s",
                                  num_cores=NC, num_subcores=NS)
    run = pl.kernel(
        _body,
        out_type=jax.ShapeDtypeStruct((OUT, B), jnp.float32),
        mesh=mesh,
        scratch_types=[
            pltpu.VMEM((B,), jnp.int32),
            pltpu.VMEM((V,), jnp.float32),
            pltpu.VMEM((H,), jnp.float32),
            pltpu.VMEM((L,), jnp.float32),
            pltpu.VMEM((L,), jnp.float32),
            pltpu.VMEM((L,), jnp.int32),
        ],
        compiler_params=pltpu.CompilerParams(use_tc_tiling_on_sc=True,
                                             needs_layout_passes=False),
    )
    out = run(bn_t, cat_ids, t2, sc, bi)
    return out.T


# speedup vs baseline: 5.6676x; 1.2835x over previous
import jax
import jax.numpy as jnp
from jax import lax
from jax.experimental import pallas as pl
from jax.experimental.pallas import tpu as pltpu
from jax.experimental.pallas import tpu_sc as plsc

B = 16384
NUM = 13
NCAT = 26
V = 100000
D = 16
OUT = NUM + NCAT * D

NC = 2
NS = 16
L = 16
NW = NC * NS
RPW = (NCAT * D) // NW
Q = B // 4
CQ = Q // L
UF = 4


def _body(bn_hbm, ids_hbm, t2_hbm, sc_hbm, bi_hbm, out_hbm,
          idx_v, row_v, ga, gb, sc_v, bi_v, flag_v,
          sem_s, sem_a, sem_b):
    wid = lax.axis_index("s") * NC + lax.axis_index("c")
    pltpu.sync_copy(sc_hbm, sc_v)
    pltpu.sync_copy(bi_hbm, bi_v)

    def stage_field(f):
        pltpu.sync_copy(ids_hbm.at[f], idx_v)

        def scan(k, acc):
            iv = idx_v[pl.ds(k * L, L)]
            return jnp.where(jnp.any(iv == 0), jnp.int32(1), acc)
        hz = lax.fori_loop(0, B // L, scan, jnp.int32(0))
        flag_v[pl.ds(0, L)] = jnp.zeros((L,), jnp.int32) + hz

    r0 = wid * RPW
    stage_field(r0 // D)
    cps = pltpu.make_async_copy(t2_hbm.at[r0 // D, r0 % D], row_v, sem_s)
    cps.start()

    @pl.when(wid < NUM)
    def _num():
        widv = jnp.full((L,), 0, jnp.int32) + wid
        s16 = plsc.load_gather(sc_v, [widv])
        b16 = plsc.load_gather(bi_v, [widv])
        for q, buf in ((0, ga), (1, gb), (2, ga), (3, gb)):
            pltpu.sync_copy(bn_hbm.at[wid, pl.ds(q * Q, Q)], buf)

            def nchunk(k, _):
                x = buf[pl.ds(k * L, L)]
                buf[pl.ds(k * L, L)] = x * s16 + b16
                return 0
            lax.fori_loop(0, CQ, nchunk, 0)
            pltpu.sync_copy(buf, out_hbm.at[wid, pl.ds(q * Q, Q)])

    cpa = None
    cpb = None
    for j in range(RPW):
        r = r0 + j
        f = r // D
        c = r + NUM
        cps.wait()
        fv = flag_v[pl.ds(0, L)]
        hz = fv[0] != 0

        for q in range(4):
            buf = ga if q % 2 == 0 else gb
            if q % 2 == 0:
                if cpa is not None:
                    cpa.wait()
            else:
                if cpb is not None:
                    cpb.wait()

            def g4(k, _):
                for u in range(UF):
                    o = (k * UF + u) * L
                    iv = idx_v[pl.ds(q * Q + o, L)]
                    buf[pl.ds(o, L)] = plsc.load_gather(row_v, [iv])
                return 0

            def g4fix(k, _):
                for u in range(UF):
                    o = (k * UF + u) * L
                    iv = idx_v[pl.ds(q * Q + o, L)]
                    vals = plsc.load_gather(row_v, [iv])
                    buf[pl.ds(o, L)] = jnp.where(iv == 0, 0.0, vals)
                return 0

            @pl.when(hz)
            def _gather_fix():
                lax.fori_loop(0, CQ // UF, g4fix, 0)

            @pl.when(jnp.logical_not(hz))
            def _gather():
                lax.fori_loop(0, CQ // UF, g4, 0)

            sem = sem_a if q % 2 == 0 else sem_b
            cp = pltpu.make_async_copy(buf, out_hbm.at[c, pl.ds(q * Q, Q)],
                                       sem)
            cp.start()
            if q % 2 == 0:
                cpa = cp
            else:
                cpb = cp

        if j + 1 < RPW:
            r2 = r0 + j + 1
            f2 = r2 // D
            cps = pltpu.make_async_copy(t2_hbm.at[f2, r2 % D], row_v, sem_s)
            cps.start()

            @pl.when(f2 != f)
            def _restage():
                stage_field(f2)

    cpa.wait()
    cpb.wait()


@jax.jit
def kernel(batch_num, cat_ids, tables, num_mean, num_std):
    t2 = jnp.transpose(tables, (0, 2, 1))
    bn_t = batch_num.T
    scale = 1.0 / num_std.reshape(NUM)
    sc = jnp.pad(scale, (0, L - NUM))
    bi = jnp.pad(-num_mean.reshape(NUM) * scale, (0, L - NUM))

    mesh = plsc.VectorSubcoreMesh(core_axis_name="c", subcore_axis_name="s",
                                  num_cores=NC, num_subcores=NS)
    run = pl.kernel(
        _body,
        out_type=jax.ShapeDtypeStruct((OUT, B), jnp.float32),
        mesh=mesh,
        scratch_types=[
            pltpu.VMEM((B,), jnp.int32),
            pltpu.VMEM((V,), jnp.float32),
            pltpu.VMEM((Q,), jnp.float32),
            pltpu.VMEM((Q,), jnp.float32),
            pltpu.VMEM((L,), jnp.float32),
            pltpu.VMEM((L,), jnp.float32),
            pltpu.VMEM((L,), jnp.int32),
            pltpu.SemaphoreType.DMA,
            pltpu.SemaphoreType.DMA,
            pltpu.SemaphoreType.DMA,
        ],
        compiler_params=pltpu.CompilerParams(use_tc_tiling_on_sc=True,
                                             needs_layout_passes=False),
    )
    out = run(bn_t, cat_ids, t2, sc, bi)
    return out.T


# speedup vs baseline: 7.5681x; 1.3353x over previous
import jax
import jax.numpy as jnp
from jax import lax
from jax.experimental import pallas as pl
from jax.experimental.pallas import tpu as pltpu
from jax.experimental.pallas import tpu_sc as plsc

B = 16384
NUM = 13
NCAT = 26
V = 100000
D = 16
OUT = NUM + NCAT * D

NC = 2
NS = 16
L = 16
NW = NC * NS
RPW = (NCAT * D) // NW
Q = B // 4
CQ = Q // L
UF = 8


def _body(bn_hbm, ids_hbm, t2_hbm, sc_hbm, bi_hbm, out_hbm,
          idx_v, row_v, ga, gb, sc_v, bi_v,
          sem_s, sem_a, sem_b):
    wid = lax.axis_index("s") * NC + lax.axis_index("c")

    def start_stream(f, d):
        c1 = pltpu.make_async_copy(t2_hbm.at[f, d], row_v, sem_s)
        c1.start()
        return c1

    r0 = wid * RPW
    cps = start_stream(r0 // D, r0 % D)
    pltpu.sync_copy(sc_hbm, sc_v)
    pltpu.sync_copy(bi_hbm, bi_v)
    pltpu.sync_copy(ids_hbm.at[r0 // D], idx_v)
    lane0 = lax.iota(jnp.int32, L) == 0

    @pl.when(wid < NUM)
    def _num():
        widv = jnp.full((L,), 0, jnp.int32) + wid
        s16 = plsc.load_gather(sc_v, [widv])
        b16 = plsc.load_gather(bi_v, [widv])
        for q, buf in ((0, ga), (1, gb), (2, ga), (3, gb)):
            pltpu.sync_copy(bn_hbm.at[wid, pl.ds(q * Q, Q)], buf)

            def nchunk(k, _):
                x = buf[pl.ds(k * L, L)]
                buf[pl.ds(k * L, L)] = x * s16 + b16
                return 0
            lax.fori_loop(0, CQ, nchunk, 0)
            pltpu.sync_copy(buf, out_hbm.at[wid, pl.ds(q * Q, Q)])

    cpa = None
    cpb = None
    for j in range(RPW):
        r = r0 + j
        f = r // D
        c = r + NUM
        cps.wait()
        row_v[pl.ds(0, L)] = jnp.where(lane0, 0.0, row_v[pl.ds(0, L)])

        for q in range(4):
            buf = ga if q % 2 == 0 else gb
            if q % 2 == 0:
                if cpa is not None:
                    cpa.wait()
            else:
                if cpb is not None:
                    cpb.wait()

            def g4(k, _):
                for u in range(UF):
                    o = (k * UF + u) * L
                    iv = idx_v[pl.ds(q * Q + o, L)]
                    buf[pl.ds(o, L)] = plsc.load_gather(row_v, [iv])
                return 0
            lax.fori_loop(0, CQ // UF, g4, 0)

            sem = sem_a if q % 2 == 0 else sem_b
            cp = pltpu.make_async_copy(buf, out_hbm.at[c, pl.ds(q * Q, Q)],
                                       sem)
            cp.start()
            if q % 2 == 0:
                cpa = cp
            else:
                cpb = cp

        if j + 1 < RPW:
            r2 = r0 + j + 1
            f2 = r2 // D
            cps = start_stream(f2, r2 % D)

            @pl.when(f2 != f)
            def _restage():
                pltpu.sync_copy(ids_hbm.at[f2], idx_v)

    cpa.wait()
    cpb.wait()


@jax.jit
def kernel(batch_num, cat_ids, tables, num_mean, num_std):
    t2 = jnp.transpose(tables, (0, 2, 1))
    bn_t = batch_num.T
    scale = 1.0 / num_std.reshape(NUM)
    sc = jnp.pad(scale, (0, L - NUM))
    bi = jnp.pad(-num_mean.reshape(NUM) * scale, (0, L - NUM))

    mesh = plsc.VectorSubcoreMesh(core_axis_name="c", subcore_axis_name="s",
                                  num_cores=NC, num_subcores=NS)
    run = pl.kernel(
        _body,
        out_type=jax.ShapeDtypeStruct((OUT, B), jnp.float32),
        mesh=mesh,
        scratch_types=[
            pltpu.VMEM((B,), jnp.int32),
            pltpu.VMEM((V,), jnp.float32),
            pltpu.VMEM((Q,), jnp.float32),
            pltpu.VMEM((Q,), jnp.float32),
            pltpu.VMEM((L,), jnp.float32),
            pltpu.VMEM((L,), jnp.float32),
            pltpu.SemaphoreType.DMA,
            pltpu.SemaphoreType.DMA,
            pltpu.SemaphoreType.DMA,
        ],
        compiler_params=pltpu.CompilerParams(use_tc_tiling_on_sc=True,
                                             needs_layout_passes=False),
    )
    out = run(bn_t, cat_ids, t2, sc, bi)
    return out.T
